# Initial kernel scaffold; baseline (speedup 1.0000x reference)
import jax
import jax.numpy as jnp
from jax.experimental import pallas as pl


def kernel(node_attr, coords, batch_id, edges, edge_type_attr, ee_w1, ee_b1, ee_w2, ee_b2, tpW0, fcW1_0, fcb1_0, fcW2_0, fcb2_0, lng_0, lnb_0, tpW1, fcW1_1, fcb1_1, fcW2_1, fcb2_1, lng_1, lnb_1, tpW2, fcW1_2, fcb1_2, fcW2_2, fcb2_2, lng_2, lnb_2, ffn_w1, ffn_b1, ffn_w2, ffn_b2):
    raise NotImplementedError("write your pallas kernel here")


# speedup vs baseline: 10.4592x; 10.4592x over previous
import functools

import jax
import jax.numpy as jnp
import numpy as np
from jax.experimental import pallas as pl
from jax.experimental.pallas import tpu as pltpu

_NS = 32
_N = 50000
_E = 800000
_DIN = (32, 56, 80)
_DOUT = (56, 80, 112)
_DP = (32, 64, 80)
_BE = 2000
_BN = 2000

_pc = pl.pallas_call


def _row(bs, c):
    return pl.BlockSpec((bs, c), lambda i: (i, 0))


def _full(shape):
    return pl.BlockSpec(shape, lambda i: tuple(0 for _ in shape))


def _geom_body(cs_ref, cd_ref, w1_ref, b1_ref, w2_ref, b2_ref, sh_ref, le_ref):
    ev = cd_ref[...] - cs_ref[...]
    x = ev[:, 0:1]
    y = ev[:, 1:2]
    z = ev[:, 2:3]
    d = jnp.sqrt(x * x + y * y + z * z + 1e-12)
    inv = 1.0 / d
    ux = x * inv
    uy = y * inv
    uz = z * inv
    s3 = float(np.sqrt(3.0))
    s5 = float(np.sqrt(5.0))
    s15 = float(np.sqrt(15.0))
    zero = jnp.zeros_like(ux)
    cols = [jnp.ones_like(ux), s3 * ux, s3 * uy, s3 * uz,
            s15 * ux * uy, s15 * uy * uz, (s5 / 2.0) * (3.0 * uz * uz - 1.0),
            s15 * ux * uz, (s15 / 2.0) * (ux * ux - uy * uy)] + [zero] * 7
    sh_ref[...] = jnp.concatenate(cols, axis=1)
    off = jax.lax.broadcasted_iota(jnp.int32, (d.shape[0], 32), 1).astype(jnp.float32) * (5.0 / 31.0)
    coeff = -0.5 / (5.0 / 31.0) ** 2
    g = jnp.exp(coeff * (d - off) ** 2)
    h = jnp.maximum(jnp.dot(g, w1_ref[...], preferred_element_type=jnp.float32)
                    + b1_ref[...], 0.0)
    le_ref[...] = jnp.dot(h, w2_ref[...], preferred_element_type=jnp.float32) + b2_ref[...]


def _edge_body(le_ref, et_ref, sg_ref, xg_ref, sh_ref,
               w1_ref, b1_ref, w2_ref, b2_ref, tp_ref, msg_ref):
    xg = xg_ref[...]
    ea = jnp.concatenate([le_ref[...], et_ref[...], sg_ref[...], xg[:, :32]], axis=1)
    h = jnp.maximum(jnp.dot(ea, w1_ref[...], preferred_element_type=jnp.float32)
                    + b1_ref[...], 0.0)
    w = jnp.dot(h, w2_ref[...], preferred_element_type=jnp.float32) + b2_ref[...]
    t = jnp.dot(xg, tp_ref[...], preferred_element_type=jnp.float32)
    sh = sh_ref[...]
    acc = sh[:, 0:1] * t[:, 0:128]
    for j in range(1, 9):
        acc = acc + sh[:, j:j + 1] * t[:, 128 * j:128 * (j + 1)]
    msg_ref[...] = acc * w


def _update_body(W, acc_ref, deg_ref, prev_ref, g_ref, b_ref, out_ref):
    u = acc_ref[...][:, :W] / jnp.maximum(deg_ref[...][:, 0:1], 1.0)
    mean = jnp.sum(u, axis=1, keepdims=True) * (1.0 / W)
    ex2 = jnp.sum(u * u, axis=1, keepdims=True) * (1.0 / W)
    r = jax.lax.rsqrt(ex2 - mean * mean + 1e-5)
    normed = (u - mean) * r * g_ref[...][:, :W] + b_ref[...][:, :W]
    out_ref[...] = prev_ref[...] + jnp.concatenate(
        [normed, jnp.zeros((normed.shape[0], 128 - W), jnp.float32)], axis=1)


def _ffn_body(nd_ref, w1_ref, b1_ref, w2_ref, b2_ref, out_ref):
    nd = nd_ref[...]
    emb = jnp.concatenate([nd[:, :32], nd[:, 80:112]], axis=1)
    h = jnp.maximum(jnp.dot(emb, w1_ref[...], preferred_element_type=jnp.float32)
                    + b1_ref[...], 0.0)
    out_ref[...] = jnp.dot(h, w2_ref[...], preferred_element_type=jnp.float32) + b2_ref[...]


def _geometry(cs, cd, ee_w1, ee_b1, ee_w2, ee_b2):
    E = cs.shape[0]
    return _pc(
        _geom_body,
        grid=(E // _BE,),
        in_specs=[_row(_BE, 16), _row(_BE, 16), _full((32, 32)), _full((1, 32)),
                  _full((32, 32)), _full((1, 32))],
        out_specs=[_row(_BE, 16), _row(_BE, 32)],
        out_shape=[jax.ShapeDtypeStruct((E, 16), jnp.float32),
                   jax.ShapeDtypeStruct((E, 32), jnp.float32)],
    )(cs, cd, ee_w1, ee_b1.reshape(1, 32), ee_w2, ee_b2.reshape(1, 32))


def _edge_dense(lemb, etype, sg, xg, sh, fcW1, fcb1, fcW2p, fcb2p, tpWp, dp):
    E = lemb.shape[0]
    return _pc(
        _edge_body,
        grid=(E // _BE,),
        in_specs=[_row(_BE, 32), _row(_BE, 32), _row(_BE, 32), _row(_BE, dp),
                  _row(_BE, 16), _full((128, 128)), _full((1, 128)),
                  _full((128, 128)), _full((1, 128)), _full((dp, 9 * 128))],
        out_specs=_row(_BE, 128),
        out_shape=jax.ShapeDtypeStruct((E, 128), jnp.float32),
    )(lemb, etype, sg, xg, sh, fcW1, fcb1, fcW2p, fcb2p, tpWp)


def _node_update(accf, deg16, prev, lng, lnb, dout):
    N = prev.shape[0]
    return _pc(
        functools.partial(_update_body, dout),
        grid=(N // _BN,),
        in_specs=[_row(_BN, 128), _row(_BN, 16), _row(_BN, 128),
                  _full((1, 128)), _full((1, 128))],
        out_specs=_row(_BN, 128),
        out_shape=jax.ShapeDtypeStruct((N, 128), jnp.float32),
    )(accf, deg16, prev, lng, lnb)


def _ffn(nd, w1, b1, w2, b2):
    N = nd.shape[0]
    return _pc(
        _ffn_body,
        grid=(N // _BN,),
        in_specs=[_row(_BN, 128), _full((64, 64)), _full((1, 64)),
                  _full((64, 32)), _full((1, 32))],
        out_specs=_row(_BN, 32),
        out_shape=jax.ShapeDtypeStruct((N, 32), jnp.float32),
    )(nd, w1, b1.reshape(1, 64), w2, b2.reshape(1, 32))


def kernel(node_attr, coords, batch_id, edges, edge_type_attr,
           ee_w1, ee_b1, ee_w2, ee_b2,
           tpW0, fcW1_0, fcb1_0, fcW2_0, fcb2_0, lng_0, lnb_0,
           tpW1, fcW1_1, fcb1_1, fcW2_1, fcb2_1, lng_1, lnb_1,
           tpW2, fcW1_2, fcb1_2, fcW2_2, fcb2_2, lng_2, lnb_2,
           ffn_w1, ffn_b1, ffn_w2, ffn_b2):
    N = node_attr.shape[0]
    E = edges.shape[1]
    f32 = jnp.float32
    src = edges[0]
    dst = edges[1]

    coords16 = jnp.pad(coords, ((0, 0), (0, 13)))
    cs = jnp.take(coords16, src, axis=0)
    cd = jnp.take(coords16, dst, axis=0)
    sh, lemb = _geometry(cs, cd, ee_w1, ee_b1, ee_w2, ee_b2)

    deg = jax.ops.segment_sum(jnp.ones((E,), f32), src, num_segments=N)
    deg16 = jnp.broadcast_to(deg.reshape(N, 1), (N, 16))

    prev = jnp.pad(node_attr, ((0, 0), (0, 128 - _NS)))
    layer_params = [(tpW0, fcW1_0, fcb1_0, fcW2_0, fcb2_0, lng_0, lnb_0),
                    (tpW1, fcW1_1, fcb1_1, fcW2_1, fcb2_1, lng_1, lnb_1),
                    (tpW2, fcW1_2, fcb1_2, fcW2_2, fcb2_2, lng_2, lnb_2)]
    for l in range(3):
        din, dout, dp = _DIN[l], _DOUT[l], _DP[l]
        tpW, fcW1, fcb1, fcW2, fcb2, lng, lnb = layer_params[l]
        tbl_d = prev[:, :dp]
        tbl_s = prev[:, :32]
        xg = jnp.take(tbl_d, dst, axis=0)
        sg = jnp.take(tbl_s, src, axis=0)
        tpWp = jnp.pad(tpW, ((0, dp - din), (0, 0), (0, 128 - dout))).reshape(dp, 9 * 128)
        fcW2p = jnp.pad(fcW2, ((0, 0), (0, 128 - dout)))
        fcb2p = jnp.pad(fcb2, (0, 128 - dout)).reshape(1, 128)
        msg = _edge_dense(lemb, edge_type_attr, sg, xg, sh,
                          fcW1, fcb1.reshape(1, 128), fcW2p, fcb2p, tpWp, dp)
        accf = jax.ops.segment_sum(msg, src, num_segments=N)
        lngp = jnp.pad(lng, (0, 128 - dout)).reshape(1, 128)
        lnbp = jnp.pad(lnb, (0, 128 - dout)).reshape(1, 128)
        prev = _node_update(accf, deg16, prev, lngp, lnbp, dout)

    return _ffn(prev, ffn_w1, ffn_b1, ffn_w2, ffn_b2)


# speedup vs baseline: 23.0754x; 2.2062x over previous
import functools

import jax
import jax.numpy as jnp
import numpy as np
from jax import lax
from jax.experimental import pallas as pl
from jax.experimental.pallas import tpu as pltpu
from jax.experimental.pallas import tpu_sc as plsc

_NS = 32
_N = 50000
_E = 800000
_DIN = (32, 56, 80)
_DOUT = (56, 80, 112)
_DP = (32, 64, 80)
_BE = 2000
_BN = 2000

_NW = 32
_EPW = _E // _NW
_NB = (_EPW + 127) // 128
_TAIL = _EPW - (_NB - 1) * 128
_NPT = _N // 16

_pc = pl.pallas_call


def _sc_mesh():
    return plsc.VectorSubcoreMesh(core_axis_name="c", subcore_axis_name="s")


def _sc_gather(tbl_d, tbl_s, gidx_d, gidx_s):
    dd = tbl_d.shape[1]
    ds = tbl_s.shape[1]

    @functools.partial(
        pl.kernel,
        mesh=_sc_mesh(),
        compiler_params=pltpu.CompilerParams(use_tc_tiling_on_sc=False),
        out_type=[jax.ShapeDtypeStruct((_E, dd), jnp.float32),
                  jax.ShapeDtypeStruct((_E, ds), jnp.float32)],
        scratch_types=[
            pltpu.VMEM((_NB, 128), jnp.int32),
            pltpu.VMEM((_NB, 128), jnp.int32),
            pltpu.VMEM((128, dd), jnp.float32),
            pltpu.VMEM((128, ds), jnp.float32),
            pltpu.SemaphoreType.DMA,
        ],
    )
    def k(tbl_d_h, tbl_s_h, gidx_d_h, gidx_s_h, xg_h, sg_h,
          idxd, idxs, bufd, bufs, sem):
        wid = lax.axis_index("s") * 2 + lax.axis_index("c")
        base = wid * _EPW
        pltpu.sync_copy(gidx_d_h.at[wid], idxd)
        pltpu.sync_copy(gidx_s_h.at[wid], idxs)

        def body(r, _):
            cd = pltpu.make_async_copy(tbl_d_h.at[idxd.at[r]], bufd, sem)
            cs = pltpu.make_async_copy(tbl_s_h.at[idxs.at[r]], bufs, sem)
            cd.start()
            cs.start()
            cd.wait()
            cs.wait()
            pltpu.sync_copy(bufd, xg_h.at[pl.ds(base + r * 128, 128)])
            pltpu.sync_copy(bufs, sg_h.at[pl.ds(base + r * 128, 128)])
            return _

        lax.fori_loop(0, _NB - 1, body, 0, unroll=False)
        r = _NB - 1
        cd = pltpu.make_async_copy(tbl_d_h.at[idxd.at[r]], bufd, sem)
        cs = pltpu.make_async_copy(tbl_s_h.at[idxs.at[r]], bufs, sem)
        cd.start()
        cs.start()
        cd.wait()
        cs.wait()
        pltpu.sync_copy(bufd.at[pl.ds(0, _TAIL)],
                        xg_h.at[pl.ds(base + r * 128, _TAIL)])
        pltpu.sync_copy(bufs.at[pl.ds(0, _TAIL)],
                        sg_h.at[pl.ds(base + r * 128, _TAIL)])

    return k(tbl_d, tbl_s, gidx_d, gidx_s)


def _sc_scatter(msg, gidx_s, zeros16, nchunk):
    @functools.partial(
        pl.kernel,
        mesh=_sc_mesh(),
        compiler_params=pltpu.CompilerParams(use_tc_tiling_on_sc=False),
        out_type=jax.ShapeDtypeStruct((_N, 128), jnp.float32),
        scratch_types=[
            pltpu.VMEM((_NB, 128), jnp.int32),
            pltpu.VMEM((_NB, 128), jnp.int32),
            pltpu.VMEM((128, 16), jnp.float32),
            pltpu.VMEM((128, 16), jnp.float32),
            pltpu.VMEM_SHARED((_N, 16), jnp.float32),
            pltpu.SemaphoreType.DMA,
        ],
    )
    def k(msg_h, gidx_h, zeros_h, out_h, idxA, idxB, buf, tbuf, acc, sem):
        cid = lax.axis_index("c")
        sid = lax.axis_index("s")
        pltpu.sync_copy(gidx_h.at[sid], idxA)
        pltpu.sync_copy(gidx_h.at[sid + 16], idxB)
        pltpu.sync_copy(zeros_h.at[pl.ds(0, 128)], tbuf)

        for j in range((nchunk + 1) // 2):
            ch = 2 * j + cid

            @pl.when(ch < nchunk)
            def _():
                pltpu.sync_copy(zeros_h.at[pl.ds(sid * _NPT, _NPT)],
                                acc.at[pl.ds(sid * _NPT, _NPT)])
                plsc.subcore_barrier()
                for wblk, idxR in ((sid, idxA), (sid + 16, idxB)):
                    ebase = wblk * _EPW

                    def body(r, _):
                        pltpu.sync_copy(
                            msg_h.at[pl.ds(ebase + r * 128, 128),
                                     pl.ds(ch * 16, 16)], buf)
                        pltpu.sync_copy(buf, acc.at[idxR.at[r]], add=True)
                        return _

                    lax.fori_loop(0, _NB - 1, body, 0, unroll=False)
                    rt = _NB - 1
                    pltpu.sync_copy(
                        msg_h.at[pl.ds(ebase + rt * 128, _TAIL),
                                 pl.ds(ch * 16, 16)],
                        tbuf.at[pl.ds(0, _TAIL)])
                    pltpu.sync_copy(tbuf, acc.at[idxR.at[rt]], add=True)
                plsc.subcore_barrier()
                pltpu.sync_copy(acc.at[pl.ds(sid * _NPT, _NPT)],
                                out_h.at[pl.ds(sid * _NPT, _NPT),
                                         pl.ds(ch * 16, 16)])

    return k(msg, gidx_s, zeros16)


def _sc_degree(gidx_s, ones_full, ones_tail, zeros16):
    @functools.partial(
        pl.kernel,
        mesh=_sc_mesh(),
        compiler_params=pltpu.CompilerParams(use_tc_tiling_on_sc=False),
        out_type=jax.ShapeDtypeStruct((_N, 16), jnp.float32),
        scratch_types=[
            pltpu.VMEM((_NB, 128), jnp.int32),
            pltpu.VMEM((_NB, 128), jnp.int32),
            pltpu.VMEM((128, 16), jnp.float32),
            pltpu.VMEM((128, 16), jnp.float32),
            pltpu.VMEM_SHARED((_N, 16), jnp.float32),
            pltpu.SemaphoreType.DMA,
        ],
    )
    def k(gidx_h, ones_h, onest_h, zeros_h, out_h,
          idxA, idxB, obuf, otbuf, acc, sem):
        cid = lax.axis_index("c")
        sid = lax.axis_index("s")

        @pl.when(cid == 0)
        def _():
            pltpu.sync_copy(gidx_h.at[sid], idxA)
            pltpu.sync_copy(gidx_h.at[sid + 16], idxB)
            pltpu.sync_copy(ones_h, obuf)
            pltpu.sync_copy(onest_h, otbuf)
            pltpu.sync_copy(zeros_h.at[pl.ds(sid * _NPT, _NPT)],
                            acc.at[pl.ds(sid * _NPT, _NPT)])
            plsc.subcore_barrier()
            for idxR in (idxA, idxB):
                def body(r, _):
                    pltpu.sync_copy(obuf, acc.at[idxR.at[r]], add=True)
                    return _

                lax.fori_loop(0, _NB - 1, body, 0, unroll=False)
                pltpu.sync_copy(otbuf, acc.at[idxR.at[_NB - 1]], add=True)
            plsc.subcore_barrier()
            pltpu.sync_copy(acc.at[pl.ds(sid * _NPT, _NPT)],
                            out_h.at[pl.ds(sid * _NPT, _NPT)])

    return k(gidx_s, ones_full, ones_tail, zeros16)


def _row(bs, c):
    return pl.BlockSpec((bs, c), lambda i: (i, 0))


def _full(shape):
    return pl.BlockSpec(shape, lambda i: tuple(0 for _ in shape))


def _geom_body(cs_ref, cd_ref, w1_ref, b1_ref, w2_ref, b2_ref, sh_ref, le_ref):
    ev = cd_ref[...] - cs_ref[...]
    x = ev[:, 0:1]
    y = ev[:, 1:2]
    z = ev[:, 2:3]
    d = jnp.sqrt(x * x + y * y + z * z + 1e-12)
    inv = 1.0 / d
    ux = x * inv
    uy = y * inv
    uz = z * inv
    s3 = float(np.sqrt(3.0))
    s5 = float(np.sqrt(5.0))
    s15 = float(np.sqrt(15.0))
    zero = jnp.zeros_like(ux)
    cols = [jnp.ones_like(ux), s3 * ux, s3 * uy, s3 * uz,
            s15 * ux * uy, s15 * uy * uz, (s5 / 2.0) * (3.0 * uz * uz - 1.0),
            s15 * ux * uz, (s15 / 2.0) * (ux * ux - uy * uy)] + [zero] * 7
    sh_ref[...] = jnp.concatenate(cols, axis=1)
    off = jax.lax.broadcasted_iota(jnp.int32, (d.shape[0], 32), 1).astype(jnp.float32) * (5.0 / 31.0)
    coeff = -0.5 / (5.0 / 31.0) ** 2
    g = jnp.exp(coeff * (d - off) ** 2)
    h = jnp.maximum(jnp.dot(g, w1_ref[...], preferred_element_type=jnp.float32)
                    + b1_ref[...], 0.0)
    le_ref[...] = jnp.dot(h, w2_ref[...], preferred_element_type=jnp.float32) + b2_ref[...]


def _edge_body(le_ref, et_ref, sg_ref, xg_ref, sh_ref,
               w1_ref, b1_ref, w2_ref, b2_ref, tp_ref, msg_ref):
    xg = xg_ref[...]
    ea = jnp.concatenate([le_ref[...], et_ref[...], sg_ref[...], xg[:, :32]], axis=1)
    h = jnp.maximum(jnp.dot(ea, w1_ref[...], preferred_element_type=jnp.float32)
                    + b1_ref[...], 0.0)
    w = jnp.dot(h, w2_ref[...], preferred_element_type=jnp.float32) + b2_ref[...]
    t = jnp.dot(xg, tp_ref[...], preferred_element_type=jnp.float32)
    sh = sh_ref[...]
    acc = sh[:, 0:1] * t[:, 0:128]
    for j in range(1, 9):
        acc = acc + sh[:, j:j + 1] * t[:, 128 * j:128 * (j + 1)]
    msg_ref[...] = acc * w


def _update_body(W, acc_ref, deg_ref, prev_ref, g_ref, b_ref, out_ref):
    u = acc_ref[...][:, :W] / jnp.maximum(deg_ref[...][:, 0:1], 1.0)
    mean = jnp.sum(u, axis=1, keepdims=True) * (1.0 / W)
    ex2 = jnp.sum(u * u, axis=1, keepdims=True) * (1.0 / W)
    r = jax.lax.rsqrt(ex2 - mean * mean + 1e-5)
    normed = (u - mean) * r * g_ref[...][:, :W] + b_ref[...][:, :W]
    out_ref[...] = prev_ref[...] + jnp.concatenate(
        [normed, jnp.zeros((normed.shape[0], 128 - W), jnp.float32)], axis=1)


def _ffn_body(nd_ref, w1_ref, b1_ref, w2_ref, b2_ref, out_ref):
    nd = nd_ref[...]
    emb = jnp.concatenate([nd[:, :32], nd[:, 80:112]], axis=1)
    h = jnp.maximum(jnp.dot(emb, w1_ref[...], preferred_element_type=jnp.float32)
                    + b1_ref[...], 0.0)
    out_ref[...] = jnp.dot(h, w2_ref[...], preferred_element_type=jnp.float32) + b2_ref[...]


def _geometry(cs, cd, ee_w1, ee_b1, ee_w2, ee_b2):
    E = cs.shape[0]
    return _pc(
        _geom_body,
        grid=(E // _BE,),
        in_specs=[_row(_BE, 16), _row(_BE, 16), _full((32, 32)), _full((1, 32)),
                  _full((32, 32)), _full((1, 32))],
        out_specs=[_row(_BE, 16), _row(_BE, 32)],
        out_shape=[jax.ShapeDtypeStruct((E, 16), jnp.float32),
                   jax.ShapeDtypeStruct((E, 32), jnp.float32)],
    )(cs, cd, ee_w1, ee_b1.reshape(1, 32), ee_w2, ee_b2.reshape(1, 32))


def _edge_dense(lemb, etype, sg, xg, sh, fcW1, fcb1, fcW2p, fcb2p, tpWp, dp):
    E = lemb.shape[0]
    return _pc(
        _edge_body,
        grid=(E // _BE,),
        in_specs=[_row(_BE, 32), _row(_BE, 32), _row(_BE, 32), _row(_BE, dp),
                  _row(_BE, 16), _full((128, 128)), _full((1, 128)),
                  _full((128, 128)), _full((1, 128)), _full((dp, 9 * 128))],
        out_specs=_row(_BE, 128),
        out_shape=jax.ShapeDtypeStruct((E, 128), jnp.float32),
    )(lemb, etype, sg, xg, sh, fcW1, fcb1, fcW2p, fcb2p, tpWp)


def _node_update(accf, deg16, prev, lng, lnb, dout):
    N = prev.shape[0]
    return _pc(
        functools.partial(_update_body, dout),
        grid=(N // _BN,),
        in_specs=[_row(_BN, 128), _row(_BN, 16), _row(_BN, 128),
                  _full((1, 128)), _full((1, 128))],
        out_specs=_row(_BN, 128),
        out_shape=jax.ShapeDtypeStruct((N, 128), jnp.float32),
    )(accf, deg16, prev, lng, lnb)


def _ffn(nd, w1, b1, w2, b2):
    N = nd.shape[0]
    return _pc(
        _ffn_body,
        grid=(N // _BN,),
        in_specs=[_row(_BN, 128), _full((64, 64)), _full((1, 64)),
                  _full((64, 32)), _full((1, 32))],
        out_specs=_row(_BN, 32),
        out_shape=jax.ShapeDtypeStruct((N, 32), jnp.float32),
    )(nd, w1, b1.reshape(1, 64), w2, b2.reshape(1, 32))


def kernel(node_attr, coords, batch_id, edges, edge_type_attr,
           ee_w1, ee_b1, ee_w2, ee_b2,
           tpW0, fcW1_0, fcb1_0, fcW2_0, fcb2_0, lng_0, lnb_0,
           tpW1, fcW1_1, fcb1_1, fcW2_1, fcb2_1, lng_1, lnb_1,
           tpW2, fcW1_2, fcb1_2, fcW2_2, fcb2_2, lng_2, lnb_2,
           ffn_w1, ffn_b1, ffn_w2, ffn_b2):
    N = node_attr.shape[0]
    E = edges.shape[1]
    f32 = jnp.float32
    src = edges[0]
    dst = edges[1]

    npad = _NB * 128 - _EPW
    padv = (jnp.arange(_NW * npad, dtype=jnp.int32) * 7919) % _N
    def _widx(ix):
        return jnp.concatenate(
            [ix.reshape(_NW, _EPW), padv.reshape(_NW, npad)],
            axis=1).reshape(_NW, _NB, 128)
    gidx_src = _widx(src)
    gidx_dst = _widx(dst)
    zeros16 = jnp.zeros((N, 16), f32)
    ones_full = jnp.ones((128, 16), f32)
    ones_tail = jnp.where(jnp.arange(128)[:, None] < _TAIL, 1.0,
                          0.0).astype(f32) * jnp.ones((128, 16), f32)

    coords16 = jnp.pad(coords, ((0, 0), (0, 13)))
    cd, cs = _sc_gather(coords16, coords16, gidx_dst, gidx_src)
    sh, lemb = _geometry(cs, cd, ee_w1, ee_b1, ee_w2, ee_b2)

    deg16 = _sc_degree(gidx_src, ones_full, ones_tail, zeros16)

    prev = jnp.pad(node_attr, ((0, 0), (0, 128 - _NS)))
    layer_params = [(tpW0, fcW1_0, fcb1_0, fcW2_0, fcb2_0, lng_0, lnb_0),
                    (tpW1, fcW1_1, fcb1_1, fcW2_1, fcb2_1, lng_1, lnb_1),
                    (tpW2, fcW1_2, fcb1_2, fcW2_2, fcb2_2, lng_2, lnb_2)]
    for l in range(3):
        din, dout, dp = _DIN[l], _DOUT[l], _DP[l]
        tpW, fcW1, fcb1, fcW2, fcb2, lng, lnb = layer_params[l]
        tbl_d = prev[:, :dp]
        tbl_s = prev[:, :32]
        xg, sg = _sc_gather(tbl_d, tbl_s, gidx_dst, gidx_src)
        tpWp = jnp.pad(tpW, ((0, dp - din), (0, 0), (0, 128 - dout))).reshape(dp, 9 * 128)
        fcW2p = jnp.pad(fcW2, ((0, 0), (0, 128 - dout)))
        fcb2p = jnp.pad(fcb2, (0, 128 - dout)).reshape(1, 128)
        msg = _edge_dense(lemb, edge_type_attr, sg, xg, sh,
                          fcW1, fcb1.reshape(1, 128), fcW2p, fcb2p, tpWp, dp)
        accf = _sc_scatter(msg, gidx_src, zeros16, (dout + 15) // 16)
        lngp = jnp.pad(lng, (0, 128 - dout)).reshape(1, 128)
        lnbp = jnp.pad(lnb, (0, 128 - dout)).reshape(1, 128)
        prev = _node_update(accf, deg16, prev, lngp, lnbp, dout)

    return _ffn(prev, ffn_w1, ffn_b1, ffn_w2, ffn_b2)


# speedup vs baseline: 28.5142x; 1.2357x over previous
import functools

import jax
import jax.numpy as jnp
import numpy as np
from jax import lax
from jax.experimental import pallas as pl
from jax.experimental.pallas import tpu as pltpu
from jax.experimental.pallas import tpu_sc as plsc

_NS = 32
_N = 50000
_E = 800000
_DIN = (32, 56, 80)
_DOUT = (56, 80, 112)
_DP = (32, 64, 96)
_BE = 2000
_BN = 2000

_NW = 32
_EPW = _E // _NW
_BS = 125
_NBB = _EPW // _BS
_G = 5
_NPT = _N // 16

_pc = pl.pallas_call


def _sc_mesh():
    return plsc.VectorSubcoreMesh(core_axis_name="c", subcore_axis_name="s")


_SC_PARAMS = dict(
    compiler_params=pltpu.CompilerParams(use_tc_tiling_on_sc=False),
)


def _sc_gather(tbl_d, tbl_s, gidx_d, gidx_s):
    dd, dt = tbl_d.shape[1], tbl_d.dtype
    ds, st = tbl_s.shape[1], tbl_s.dtype

    @functools.partial(
        pl.kernel,
        mesh=_sc_mesh(),
        out_type=[jax.ShapeDtypeStruct((_E, dd), dt),
                  jax.ShapeDtypeStruct((_E, ds), st)],
        scratch_types=[
            pltpu.VMEM((_NBB, _BS), jnp.int32),
            pltpu.VMEM((_NBB, _BS), jnp.int32),
            pltpu.VMEM((2, _BS, dd), dt),
            pltpu.VMEM((2, _BS, ds), st),
            pltpu.SemaphoreType.DMA,
            pltpu.SemaphoreType.DMA,
            pltpu.SemaphoreType.DMA,
            pltpu.SemaphoreType.DMA,
        ],
        **_SC_PARAMS,
    )
    def k(tbl_d_h, tbl_s_h, gidx_d_h, gidx_s_h, xg_h, sg_h,
          idxd, idxs, bufd, bufs, semg0, semg1, semw0, semw1):
        wid = lax.axis_index("s") * 2 + lax.axis_index("c")
        base = wid * _EPW
        pltpu.sync_copy(gidx_d_h.at[wid], idxd)
        pltpu.sync_copy(gidx_s_h.at[wid], idxs)
        semg = (semg0, semg1)
        semw = (semw0, semw1)

        def fire(r, s):
            pltpu.async_copy(tbl_d_h.at[idxd.at[r]], bufd.at[s], semg[s])
            pltpu.async_copy(tbl_s_h.at[idxs.at[r]], bufs.at[s], semg[s])

        def wait_g(s):
            pltpu.make_async_copy(tbl_d_h.at[idxd.at[0]], bufd.at[s], semg[s]).wait()
            pltpu.make_async_copy(tbl_s_h.at[idxs.at[0]], bufs.at[s], semg[s]).wait()

        def write(r, s):
            pltpu.async_copy(bufd.at[s], xg_h.at[pl.ds(base + r * _BS, _BS)], semw[s])
            pltpu.async_copy(bufs.at[s], sg_h.at[pl.ds(base + r * _BS, _BS)], semw[s])

        def wait_w(s):
            pltpu.make_async_copy(bufd.at[s], xg_h.at[pl.ds(base, _BS)], semw[s]).wait()
            pltpu.make_async_copy(bufs.at[s], sg_h.at[pl.ds(base, _BS)], semw[s]).wait()

        fire(0, 0)

        def body(i, carry):
            r0 = 2 * i
            r1 = r0 + 1
            wait_g(0)
            fire(r1, 1)
            write(r0, 0)
            wait_w(0)
            wait_g(1)

            @pl.when(i < _NBB // 2 - 1)
            def _():
                fire(r0 + 2, 0)

            write(r1, 1)
            wait_w(1)
            return carry

        lax.fori_loop(0, _NBB // 2, body, 0, unroll=False)

    return k(tbl_d, tbl_s, gidx_d, gidx_s)


def _sc_scatter(msg, gidx_s, zeros32, nch):
    @functools.partial(
        pl.kernel,
        mesh=_sc_mesh(),
        out_type=jax.ShapeDtypeStruct((_N, 128), jnp.float32),
        scratch_types=[
            pltpu.VMEM((2 * _NBB, _BS), jnp.int32),
            pltpu.VMEM((2, _G, _BS, 16), jnp.float32),
            pltpu.VMEM_SHARED((_N, 16), jnp.float32),
            pltpu.SemaphoreType.DMA,
            pltpu.SemaphoreType.DMA,
            pltpu.SemaphoreType.DMA,
            pltpu.SemaphoreType.DMA,
        ],
        **_SC_PARAMS,
    )
    def k(msg_h, gidx_h, zeros_h, out_h, idx2, ring, acc,
          semr0, semr1, sema0, sema1):
        cid = lax.axis_index("c")
        sid = lax.axis_index("s")
        pltpu.sync_copy(gidx_h.at[sid], idx2.at[pl.ds(0, _NBB)])
        pltpu.sync_copy(gidx_h.at[sid + 16], idx2.at[pl.ds(_NBB, _NBB)])
        semr = (semr0, semr1)
        sema = (sema0, sema1)
        ngrp = 2 * _NBB // _G

        for j in range((nch + 1) // 2):
            ch = 2 * j + cid

            @pl.when(ch < nch)
            def _():
                pltpu.sync_copy(zeros_h.at[pl.ds(sid * _NPT, _NPT), pl.ds(0, 16)],
                                acc.at[pl.ds(sid * _NPT, _NPT)])
                plsc.subcore_barrier()

                def rd_src(b):
                    ebase = jnp.where(
                        b < _NBB,
                        sid * _EPW + b * _BS,
                        (sid + 16) * _EPW + (b - _NBB) * _BS)
                    return msg_h.at[pl.ds(ebase, _BS), pl.ds(ch * 16, 16)]

                def fire_reads(g, s):
                    for i in range(_G):
                        pltpu.async_copy(rd_src(g * _G + i), ring.at[s, i],
                                         semr[s])

                def wait_reads(s):
                    for i in range(_G):
                        pltpu.make_async_copy(rd_src(0), ring.at[s, i],
                                              semr[s]).wait()

                def fire_adds(g, s):
                    for i in range(_G):
                        b = g * _G + i
                        pltpu.async_copy(ring.at[s, i], acc.at[idx2.at[b]],
                                         sema[s], add=True)

                def wait_adds(s):
                    for i in range(_G):
                        pltpu.make_async_copy(ring.at[s, i],
                                              acc.at[idx2.at[0]],
                                              sema[s]).wait()

                fire_reads(0, 0)

                def body(i, carry):
                    g0 = 2 * i
                    g1 = g0 + 1
                    wait_reads(0)

                    @pl.when(i > 0)
                    def _():
                        wait_adds(1)

                    fire_reads(g1, 1)
                    fire_adds(g0, 0)
                    wait_reads(1)
                    wait_adds(0)

                    @pl.when(i < ngrp // 2 - 1)
                    def _():
                        fire_reads(g0 + 2, 0)

                    fire_adds(g1, 1)
                    return carry

                lax.fori_loop(0, ngrp // 2, body, 0, unroll=False)
                wait_adds(1)
                plsc.subcore_barrier()
                pltpu.sync_copy(acc.at[pl.ds(sid * _NPT, _NPT)],
                                out_h.at[pl.ds(sid * _NPT, _NPT),
                                         pl.ds(ch * 16, 16)])

    return k(msg, gidx_s, zeros32)


def _sc_degree(gidx_s, ones_b, zeros32):
    @functools.partial(
        pl.kernel,
        mesh=_sc_mesh(),
        out_type=jax.ShapeDtypeStruct((_N, 16), jnp.float32),
        scratch_types=[
            pltpu.VMEM((2 * _NBB, _BS), jnp.int32),
            pltpu.VMEM((_BS, 16), jnp.float32),
            pltpu.VMEM_SHARED((_N, 16), jnp.float32),
            pltpu.SemaphoreType.DMA,
        ],
        **_SC_PARAMS,
    )
    def k(gidx_h, ones_h, zeros_h, out_h, idx2, obuf, acc, sem):
        cid = lax.axis_index("c")
        sid = lax.axis_index("s")

        @pl.when(cid == 0)
        def _():
            pltpu.sync_copy(gidx_h.at[sid], idx2.at[pl.ds(0, _NBB)])
            pltpu.sync_copy(gidx_h.at[sid + 16], idx2.at[pl.ds(_NBB, _NBB)])
            pltpu.sync_copy(ones_h, obuf)
            pltpu.sync_copy(
                zeros_h.at[pl.ds(sid * _NPT, _NPT), pl.ds(0, 16)],
                acc.at[pl.ds(sid * _NPT, _NPT)])
            plsc.subcore_barrier()

            def body(gi, carry):
                for i in range(_G):
                    b = gi * _G + i
                    pltpu.async_copy(obuf, acc.at[idx2.at[b]], sem, add=True)
                for i in range(_G):
                    pltpu.make_async_copy(obuf, acc.at[idx2.at[0]], sem).wait()
                return carry

            lax.fori_loop(0, 2 * _NBB // _G, body, 0, unroll=False)
            plsc.subcore_barrier()
            pltpu.sync_copy(acc.at[pl.ds(sid * _NPT, _NPT)],
                            out_h.at[pl.ds(sid * _NPT, _NPT)])

    return k(gidx_s, ones_b, zeros32)


def _geom_body(cs_ref, cd_ref, shm_ref, w1_ref, b1_ref, w2_ref, b2_ref,
               sh_ref, le_ref):
    ev = cd_ref[...] - cs_ref[...]
    x = ev[:, 0:1]
    y = ev[:, 1:2]
    z = ev[:, 2:3]
    d = jnp.sqrt(x * x + y * y + z * z + 1e-12)
    inv = 1.0 / d
    B = ev.shape[0]
    ux = jnp.broadcast_to(x * inv, (B, 16))
    uy = jnp.broadcast_to(y * inv, (B, 16))
    uz = jnp.broadcast_to(z * inv, (B, 16))
    m = shm_ref[...]
    sh = (m[0:1, :]
          + m[1:2, :] * ux
          + m[2:3, :] * uy
          + m[3:4, :] * uz
          + m[4:5, :] * (ux * uy)
          + m[5:6, :] * (uy * uz)
          + m[6:7, :] * (ux * uz)
          + m[7:8, :] * (ux * ux)
          + m[8:9, :] * (uy * uy)
          + m[9:10, :] * (uz * uz))
    sh_ref[...] = sh
    off = jax.lax.broadcasted_iota(
        jnp.int32, (d.shape[0], 32), 1).astype(jnp.float32) * (5.0 / 31.0)
    coeff = -0.5 / (5.0 / 31.0) ** 2
    g = jnp.exp(coeff * (d - off) ** 2)
    h = jnp.maximum(jnp.dot(g, w1_ref[...], preferred_element_type=jnp.float32)
                    + b1_ref[...], 0.0)
    le_ref[...] = jnp.dot(h, w2_ref[...], preferred_element_type=jnp.float32) + b2_ref[...]


def _edge_body(le_ref, et_ref, sg_ref, xg_ref, sh_ref,
               w1_ref, b1_ref, w2_ref, b2_ref, tp_ref, msg_ref):
    bf16 = jnp.bfloat16
    xg = xg_ref[...]
    ea = jnp.concatenate([le_ref[...].astype(bf16), et_ref[...].astype(bf16),
                          sg_ref[...], xg[:, :32]], axis=1)
    h = jnp.maximum(jnp.dot(ea, w1_ref[...],
                            preferred_element_type=jnp.float32) + b1_ref[...], 0.0)
    w = jnp.dot(h.astype(bf16), w2_ref[...],
                preferred_element_type=jnp.float32) + b2_ref[...]
    t = jnp.dot(xg, tp_ref[...],
                preferred_element_type=jnp.float32)
    sh = sh_ref[...]
    acc = sh[:, 0:1] * t[:, 0:128]
    for j in range(1, 9):
        acc = acc + sh[:, j:j + 1] * t[:, 128 * j:128 * (j + 1)]
    msg_ref[...] = acc * w


def _update_body(W, DPN, acc_ref, deg_ref, prev_ref, g_ref, b_ref,
                 out_ref, td_ref=None, ts_ref=None):
    u = acc_ref[...][:, :W] / jnp.maximum(deg_ref[...][:, 0:1], 1.0)
    mean = jnp.sum(u, axis=1, keepdims=True) * (1.0 / W)
    ex2 = jnp.sum(u * u, axis=1, keepdims=True) * (1.0 / W)
    r = jax.lax.rsqrt(ex2 - mean * mean + 1e-5)
    normed = (u - mean) * r * g_ref[...][:, :W] + b_ref[...][:, :W]
    new = prev_ref[...] + jnp.concatenate(
        [normed, jnp.zeros((normed.shape[0], 128 - W), jnp.float32)], axis=1)
    out_ref[...] = new
    if td_ref is not None:
        td_ref[...] = new[:, :DPN].astype(jnp.bfloat16)
        ts_ref[...] = new[:, :32].astype(jnp.bfloat16)


def _ffn_body(nd_ref, w1_ref, b1_ref, w2_ref, b2_ref, out_ref):
    nd = nd_ref[...]
    emb = jnp.concatenate([nd[:, :32], nd[:, 80:112]], axis=1)
    h = jnp.maximum(jnp.dot(emb, w1_ref[...], preferred_element_type=jnp.float32)
                    + b1_ref[...], 0.0)
    out_ref[...] = jnp.dot(h, w2_ref[...], preferred_element_type=jnp.float32) + b2_ref[...]


def _row(bs, c):
    return pl.BlockSpec((bs, c), lambda i: (i, 0))


def _full(shape):
    return pl.BlockSpec(shape, lambda i: tuple(0 for _ in shape))


def _sh_masks():
    s3, s5, s15 = np.sqrt(3.0), np.sqrt(5.0), np.sqrt(15.0)
    m = np.zeros((16, 16), np.float32)
    m[0, 0] = 1.0
    m[0, 6] = -s5 / 2.0
    m[1, 1] = s3
    m[2, 2] = s3
    m[3, 3] = s3
    m[4, 4] = s15
    m[5, 5] = s15
    m[6, 7] = s15
    m[7, 8] = s15 / 2.0
    m[8, 8] = -s15 / 2.0
    m[9, 6] = 1.5 * s5
    return jnp.asarray(m)


def _geometry(cs, cd, ee_w1, ee_b1, ee_w2, ee_b2):
    E = cs.shape[0]
    return _pc(
        _geom_body,
        grid=(E // _BE,),
        in_specs=[_row(_BE, 16), _row(_BE, 16), _full((16, 16)),
                  _full((32, 32)), _full((1, 32)),
                  _full((32, 32)), _full((1, 32))],
        out_specs=[_row(_BE, 16), _row(_BE, 32)],
        out_shape=[jax.ShapeDtypeStruct((E, 16), jnp.float32),
                   jax.ShapeDtypeStruct((E, 32), jnp.float32)],
    )(cs, cd, _sh_masks(), ee_w1, ee_b1.reshape(1, 32),
      ee_w2, ee_b2.reshape(1, 32))


def _edge_dense(lemb, etype, sg, xg, sh, fcW1, fcb1, fcW2p, fcb2p, tpWp, dp):
    E = lemb.shape[0]
    return _pc(
        _edge_body,
        grid=(E // _BE,),
        in_specs=[_row(_BE, 32), _row(_BE, 32), _row(_BE, 32), _row(_BE, dp),
                  _row(_BE, 16), _full((128, 128)), _full((1, 128)),
                  _full((128, 128)), _full((1, 128)), _full((dp, 9 * 128))],
        out_specs=_row(_BE, 128),
        out_shape=jax.ShapeDtypeStruct((E, 128), jnp.float32),
    )(lemb, etype, sg, xg, sh, fcW1, fcb1, fcW2p, fcb2p, tpWp)


def _node_update(accf, deg16, prev, lng, lnb, dout, dpn):
    N = prev.shape[0]
    if dpn is None:
        body = functools.partial(_update_body, dout, None)
        out_specs = _row(_BN, 128)
        out_shape = jax.ShapeDtypeStruct((N, 128), jnp.float32)
    else:
        body = functools.partial(_update_body, dout, dpn)
        out_specs = [_row(_BN, 128), _row(_BN, dpn), _row(_BN, 32)]
        out_shape = [jax.ShapeDtypeStruct((N, 128), jnp.float32),
                     jax.ShapeDtypeStruct((N, dpn), jnp.bfloat16),
                     jax.ShapeDtypeStruct((N, 32), jnp.bfloat16)]
    return _pc(
        body,
        grid=(N // _BN,),
        in_specs=[_row(_BN, 128), _row(_BN, 16), _row(_BN, 128),
                  _full((1, 128)), _full((1, 128))],
        out_specs=out_specs,
        out_shape=out_shape,
    )(accf, deg16, prev, lng, lnb)


def _ffn(nd, w1, b1, w2, b2):
    N = nd.shape[0]
    return _pc(
        _ffn_body,
        grid=(N // _BN,),
        in_specs=[_row(_BN, 128), _full((64, 64)), _full((1, 64)),
                  _full((64, 32)), _full((1, 32))],
        out_specs=_row(_BN, 32),
        out_shape=jax.ShapeDtypeStruct((N, 32), jnp.float32),
    )(nd, w1, b1.reshape(1, 64), w2, b2.reshape(1, 32))


def kernel(node_attr, coords, batch_id, edges, edge_type_attr,
           ee_w1, ee_b1, ee_w2, ee_b2,
           tpW0, fcW1_0, fcb1_0, fcW2_0, fcb2_0, lng_0, lnb_0,
           tpW1, fcW1_1, fcb1_1, fcW2_1, fcb2_1, lng_1, lnb_1,
           tpW2, fcW1_2, fcb1_2, fcW2_2, fcb2_2, lng_2, lnb_2,
           ffn_w1, ffn_b1, ffn_w2, ffn_b2):
    N = node_attr.shape[0]
    f32 = jnp.float32
    bf16 = jnp.bfloat16
    src = edges[0]
    dst = edges[1]

    gidx_src = src.reshape(_NW, _NBB, _BS)
    gidx_dst = dst.reshape(_NW, _NBB, _BS)
    zeros32 = jnp.zeros((N, 32), f32)
    ones_b = jnp.ones((_BS, 16), f32)

    coords16 = jnp.pad(coords, ((0, 0), (0, 13)))
    cd, cs = _sc_gather(coords16, coords16, gidx_dst, gidx_src)
    sh, lemb = _geometry(cs, cd, ee_w1, ee_b1, ee_w2, ee_b2)

    deg16 = _sc_degree(gidx_src, ones_b, zeros32)

    prev = jnp.pad(node_attr, ((0, 0), (0, 128 - _NS)))
    tbl_d = node_attr.astype(bf16)
    tbl_s = tbl_d
    layer_params = [(tpW0, fcW1_0, fcb1_0, fcW2_0, fcb2_0, lng_0, lnb_0),
                    (tpW1, fcW1_1, fcb1_1, fcW2_1, fcb2_1, lng_1, lnb_1),
                    (tpW2, fcW1_2, fcb1_2, fcW2_2, fcb2_2, lng_2, lnb_2)]
    for l in range(3):
        din, dout, dp = _DIN[l], _DOUT[l], _DP[l]
        tpW, fcW1, fcb1, fcW2, fcb2, lng, lnb = layer_params[l]
        xg, sg = _sc_gather(tbl_d, tbl_s, gidx_dst, gidx_src)
        tpWp = jnp.pad(tpW, ((0, dp - din), (0, 0),
                             (0, 128 - dout))).reshape(dp, 9 * 128).astype(bf16)
        fcW2p = jnp.pad(fcW2, ((0, 0), (0, 128 - dout))).astype(bf16)
        fcb2p = jnp.pad(fcb2, (0, 128 - dout)).reshape(1, 128)
        msg = _edge_dense(lemb, edge_type_attr, sg, xg, sh,
                          fcW1.astype(bf16), fcb1.reshape(1, 128), fcW2p,
                          fcb2p, tpWp, dp)
        accf = _sc_scatter(msg, gidx_src, zeros32, (dout + 15) // 16)
        lngp = jnp.pad(lng, (0, 128 - dout)).reshape(1, 128)
        lnbp = jnp.pad(lnb, (0, 128 - dout)).reshape(1, 128)
        if l < 2:
            prev, tbl_d, tbl_s = _node_update(accf, deg16, prev, lngp, lnbp,
                                              dout, _DP[l + 1])
        else:
            prev = _node_update(accf, deg16, prev, lngp, lnbp, dout, None)

    return _ffn(prev, ffn_w1, ffn_b1, ffn_w2, ffn_b2)
